# baseline (speedup 1.0000x reference)
import functools

import jax
import jax.numpy as jnp
from jax import lax
from jax.experimental import pallas as pl
from jax.experimental.pallas import tpu as pltpu
from jax.experimental.pallas import tpu_sc as plsc

B, S, T = 4, 2048, 1024
N = B * S
NC, NS, L = 2, 16, 16
NW = NC * NS
ROWS_PER_W = N // NW
G = 16
NG = ROWS_PER_W // G
NB = 4
NACC = 4
CHUNKS = T // L
CPA = CHUNKS // NACC
SEG = CPA * L
RU = 2

_mesh = plsc.VectorSubcoreMesh(core_axis_name="c", subcore_axis_name="s")


@functools.partial(
    pl.kernel,
    mesh=_mesh,
    out_type=jax.ShapeDtypeStruct((N, T), jnp.float32),
    scratch_types=[
        pltpu.VMEM((G, T), jnp.float32),
        pltpu.VMEM((G, T), jnp.float32),
        pltpu.VMEM((G, T), jnp.float32),
        pltpu.VMEM((G, T), jnp.float32),
        pltpu.VMEM((G, T), jnp.float32),
        pltpu.VMEM((G, T), jnp.float32),
        pltpu.VMEM((G,), jnp.int32),
        pltpu.VMEM((G,), jnp.int32),
        pltpu.SemaphoreType.DMA,
        pltpu.SemaphoreType.DMA,
        pltpu.SemaphoreType.DMA,
    ],
    compiler_params=pltpu.CompilerParams(needs_layout_passes=False),
)
def _crf_head(x_hbm, t_hbm, out_hbm, xb0, xb1, xb2, xb3, tb0, tb1,
              ib0, ib1, in_sem, g_sem, out_sem):
    xb = (xb0, xb1, xb2, xb3)
    tb = (tb0, tb1)
    ib = (ib0, ib1)
    wid = lax.axis_index("s") * NC + lax.axis_index("c")
    base = wid * ROWS_PER_W
    lane = lax.iota(jnp.int32, L)
    col_rel = [lane + c * L for c in range(CPA)]

    def start_in(g, b):
        pltpu.async_copy(x_hbm.at[pl.ds(base + g * G, G)], xb[b], in_sem)

    def wait_in(b):
        pltpu.make_async_copy(x_hbm.at[pl.ds(0, G)], xb[b], in_sem).wait()

    def argmax_rows(x_v, rows):
        nr = len(rows)
        m = [[jnp.full((L,), -jnp.inf, jnp.float32)] * NACC
             for _ in range(nr)]
        for c in range(CPA):
            for a in range(NACC):
                ch = a * CPA + c
                for k in range(nr):
                    m[k][a] = jnp.maximum(m[k][a],
                                          x_v[rows[k], pl.ds(ch * L, L)])
        cmins = []
        for k in range(nr):
            mm = m[k][0]
            ach = jnp.zeros((L,), jnp.int32)
            for a in range(1, NACC):
                cmp = m[k][a] > mm
                mm = jnp.where(cmp, m[k][a], mm)
                ach = jnp.where(cmp, jnp.full((L,), a, jnp.int32), ach)
            ms = jnp.max(mm)
            msv = jnp.full((L,), ms)
            astar = jnp.min(jnp.where(mm == msv, ach,
                                      jnp.full((L,), NACC, jnp.int32)))
            seg = astar * SEG
            macc = jnp.full((L,), T, jnp.int32)
            for c in range(CPA):
                v = x_v[rows[k], pl.ds(seg + c * L, L)]
                macc = jnp.minimum(macc,
                                   jnp.where(v == msv, col_rel[c],
                                             jnp.full((L,), T, jnp.int32)))
            cmins.append(seg + jnp.min(macc))
        return cmins

    def argmax(b, i):
        x_v = xb[b]

        def row_body(rr, ivec):
            rows = [rr * RU + k for k in range(RU)]
            cmins = argmax_rows(x_v, rows)
            for k in range(RU):
                ivec = jnp.where(lane == rows[k],
                                 jnp.full((L,), cmins[k]), ivec)
            return ivec

        ib[i][...] = lax.fori_loop(0, G // RU, row_body,
                                   jnp.zeros((L,), jnp.int32))

    def start_gather(i):
        pltpu.async_copy(t_hbm.at[ib[i]], tb[i], g_sem)

    def wait_gather(i):
        pltpu.make_async_copy(t_hbm.at[ib[i]], tb[i], g_sem).wait()

    def add(b, i):
        x_v, t_v = xb[b], tb[i]

        def row_body(r, carry):
            for c in range(CHUNKS):
                off = c * L
                plsc.addupdate(x_v.at[r, pl.ds(off, L)], t_v[r, pl.ds(off, L)])
            return carry

        lax.fori_loop(0, G, row_body, 0)

    def start_out(g, b):
        pltpu.async_copy(xb[b], out_hbm.at[pl.ds(base + g * G, G)], out_sem)

    def wait_out(b):
        pltpu.make_async_copy(xb[b], out_hbm.at[pl.ds(0, G)], out_sem).wait()

    start_in(0, 0)
    start_in(1, 1)
    start_in(2, 2)
    wait_in(0)
    argmax(0, 0)
    start_gather(0)

    def outer(o, carry):
        for b in range(NB):
            g = o * NB + b
            i = b % 2

            @pl.when(g + 1 < NG)
            def _():
                wait_in((b + 1) % NB)
                argmax((b + 1) % NB, (i + 1) % 2)
                start_gather((i + 1) % 2)

            @pl.when(jnp.logical_and(g >= 1, g + 3 < NG))
            def _():
                wait_out((b + 3) % NB)

            @pl.when(g + 3 < NG)
            def _():
                start_in(g + 3, (b + 3) % NB)

            wait_gather(i)
            add(b, i)
            start_out(g, b)
        return carry

    lax.fori_loop(0, NG // NB, outer, 0)
    for b in range(NB):
        wait_out(b)


def kernel(launch_matrix, transitions):
    x = launch_matrix.reshape(N, T)
    out = _crf_head(x, transitions)
    return out.reshape(B, S, T)


# speedup vs baseline: 1.7952x; 1.7952x over previous
import functools

import jax
import jax.numpy as jnp
from jax import lax
from jax.experimental import pallas as pl
from jax.experimental.pallas import tpu as pltpu
from jax.experimental.pallas import tpu_sc as plsc

B, S, T = 4, 2048, 1024
N = B * S
NC, NS, L = 2, 16, 16
NW = NC * NS
ROWS_PER_W = N // NW
G = 16
NG = ROWS_PER_W // G
NB = 4
NACC = 4
CHUNKS = T // L
CPA = CHUNKS // NACC
SEG = CPA * L

_mesh = plsc.VectorSubcoreMesh(core_axis_name="c", subcore_axis_name="s")


@functools.partial(
    pl.kernel,
    mesh=_mesh,
    out_type=jax.ShapeDtypeStruct((N, T), jnp.float32),
    scratch_types=[
        pltpu.VMEM((G, T), jnp.float32),
        pltpu.VMEM((G, T), jnp.float32),
        pltpu.VMEM((G, T), jnp.float32),
        pltpu.VMEM((G, T), jnp.float32),
        pltpu.VMEM((G,), jnp.int32),
        pltpu.VMEM((G,), jnp.int32),
        pltpu.SemaphoreType.DMA,
        pltpu.SemaphoreType.DMA,
        pltpu.SemaphoreType.DMA,
    ],
    compiler_params=pltpu.CompilerParams(needs_layout_passes=False),
)
def _crf_head(x_hbm, t_hbm, out_hbm, xb0, xb1, xb2, xb3,
              ib0, ib1, in_sem, g_sem, out_sem):
    xb = (xb0, xb1, xb2, xb3)
    ib = (ib0, ib1)
    wid = lax.axis_index("s") * NC + lax.axis_index("c")
    base = wid * ROWS_PER_W
    lane = lax.iota(jnp.int32, L)
    col_rel = [lane + c * L for c in range(CPA)]

    def start_in(g, b):
        pltpu.async_copy(x_hbm.at[pl.ds(base + g * G, G)], xb[b], in_sem)

    def wait_in(b):
        pltpu.make_async_copy(x_hbm.at[pl.ds(0, G)], xb[b], in_sem).wait()

    def argmax(b, i):
        x_v = xb[b]

        def row_body(r, ivec):
            m = [jnp.full((L,), -jnp.inf, jnp.float32)] * NACC
            for c in range(CPA):
                for a in range(NACC):
                    ch = a * CPA + c
                    m[a] = jnp.maximum(m[a], x_v[r, pl.ds(ch * L, L)])
            mm = m[0]
            ach = jnp.zeros((L,), jnp.int32)
            for a in range(1, NACC):
                cmp = m[a] > mm
                mm = jnp.where(cmp, m[a], mm)
                ach = jnp.where(cmp, jnp.full((L,), a, jnp.int32), ach)
            ms = jnp.max(mm)
            msv = jnp.full((L,), ms)
            astar = jnp.min(jnp.where(mm == msv, ach,
                                      jnp.full((L,), NACC, jnp.int32)))
            seg = astar * SEG
            macc = jnp.full((L,), T, jnp.int32)
            for c in range(CPA):
                v = x_v[r, pl.ds(seg + c * L, L)]
                macc = jnp.minimum(macc, jnp.where(v == msv, col_rel[c],
                                                   jnp.full((L,), T,
                                                            jnp.int32)))
            cmin = seg + jnp.min(macc)
            return jnp.where(lane == r, jnp.full((L,), cmin), ivec)

        ib[i][...] = lax.fori_loop(0, G, row_body,
                                   jnp.zeros((L,), jnp.int32))

    def start_gather(b, i):
        pltpu.async_copy(t_hbm.at[ib[i]], xb[b], g_sem, add=True)

    def wait_gather(b, i):
        pltpu.make_async_copy(t_hbm.at[ib[i]], xb[b], g_sem).wait()

    def start_out(g, b):
        pltpu.async_copy(xb[b], out_hbm.at[pl.ds(base + g * G, G)], out_sem)

    def wait_out(b):
        pltpu.make_async_copy(xb[b], out_hbm.at[pl.ds(0, G)], out_sem).wait()

    start_in(0, 0)

    def outer(o, carry):
        for b in range(NB):
            g = o * NB + b
            i = b % 2

            @pl.when(jnp.logical_and(g >= 3, g + 1 < NG))
            def _():
                wait_out((b + 1) % NB)

            @pl.when(g + 1 < NG)
            def _():
                start_in(g + 1, (b + 1) % NB)

            wait_in(b)
            argmax(b, i)

            @pl.when(g >= 1)
            def _():
                wait_gather((b + 3) % NB, (i + 1) % 2)
                start_out(g - 1, (b + 3) % NB)

            start_gather(b, i)
        return carry

    lax.fori_loop(0, NG // NB, outer, 0)

    wait_gather(3, 1)
    start_out(NG - 1, 3)
    for b in range(NB):
        wait_out(b)


def kernel(launch_matrix, transitions):
    x = launch_matrix.reshape(N, T)
    out = _crf_head(x, transitions)
    return out.reshape(B, S, T)
